# baseline (speedup 1.0000x reference)
import jax
import jax.numpy as jnp
from jax.experimental import pallas as pl
from jax.experimental.pallas import tpu as pltpu

_A = 25
_S = 5
_L = _S * _A
_BT = 2048


def _blosum_tile(yt_ref, m_ref, yp_ref, r_ref, bd_ref, num_ref, den_ref):
    yt5 = yt_ref[...]
    m5 = m_ref[...]
    yp5 = yp_ref[...]
    r = r_ref[...]
    bd = bd_ref[...]

    ytx = jnp.dot(yt5, r, preferred_element_type=jnp.float32)
    mxp = jnp.dot(m5, r, preferred_element_type=jnp.float32)
    cls = (jax.lax.broadcasted_iota(jnp.int32, (_BT, _L), 1) % _A
           ).astype(jnp.float32)
    ohm = jnp.where(ytx == cls, mxp, 0.0)
    w = jnp.dot(ohm, bd, preferred_element_type=jnp.float32)
    num_ref[...] = jnp.sum(yp5 * w).reshape(1, 1, 1)
    den_ref[...] = jnp.sum(m5).reshape(1, 1, 1)


def kernel(y_true, y_pred, mask, B):
    n = y_true.shape[0] * y_true.shape[1]
    rows = n // _S
    tiles = rows // _BT
    yt5 = y_true.reshape(rows, _S).astype(jnp.float32)
    m5 = mask.reshape(rows, _S)
    yp5 = y_pred.reshape(rows, _L)
    r = (jnp.arange(_S)[:, None] == (jnp.arange(_L)[None, :] // _A)
         ).astype(jnp.float32)
    bd = jnp.kron(jnp.eye(_S, dtype=jnp.float32), B)

    num, den = pl.pallas_call(
        _blosum_tile,
        grid=(tiles,),
        in_specs=[
            pl.BlockSpec((_BT, _S), lambda i: (i, 0)),
            pl.BlockSpec((_BT, _S), lambda i: (i, 0)),
            pl.BlockSpec((_BT, _L), lambda i: (i, 0)),
            pl.BlockSpec((_S, _L), lambda i: (0, 0)),
            pl.BlockSpec((_L, _L), lambda i: (0, 0)),
        ],
        out_specs=[
            pl.BlockSpec((1, 1, 1), lambda i: (i, 0, 0)),
            pl.BlockSpec((1, 1, 1), lambda i: (i, 0, 0)),
        ],
        out_shape=[
            jax.ShapeDtypeStruct((tiles, 1, 1), jnp.float32),
            jax.ShapeDtypeStruct((tiles, 1, 1), jnp.float32),
        ],
        compiler_params=pltpu.CompilerParams(
            dimension_semantics=("parallel",),
        ),
    )(yt5, m5, yp5, r, bd)
    return jnp.sum(num) / jnp.sum(den)
